# Initial kernel scaffold; baseline (speedup 1.0000x reference)
import jax
import jax.numpy as jnp
from jax.experimental import pallas as pl


def kernel(x, weight):
    raise NotImplementedError("write your pallas kernel here")


# speedup vs baseline: 2.5920x; 2.5920x over previous
import functools

import jax
import jax.numpy as jnp
from jax.experimental import pallas as pl
from jax.experimental.pallas import tpu as pltpu

_DIM = 2048
_E = 64
_K = 8
_G = 8
_KG = 4
_GS = _E // _G


def _gate_block(x_ref, w_ref, w_out_ref, i_out_ref):
    x = x_ref[...]
    w = w_ref[...]
    logits = jax.lax.dot_general(
        x, w, (((1,), (1,)), ((), ())), preferred_element_type=jnp.float32)
    s = jax.nn.sigmoid(logits)
    bt = s.shape[0]
    col = jax.lax.broadcasted_iota(jnp.int32, (bt, _E), 1)
    gcol = col // _GS
    neg = jnp.float32(-jnp.inf)

    gmb = jnp.full((bt, _E), neg, jnp.float32)
    for g in range(_G):
        in_g = gcol == g
        gm = jnp.max(jnp.where(in_g, s, neg), axis=1, keepdims=True)
        gmb = jnp.where(in_g, gm, gmb)

    avail = jnp.ones((bt, _E), jnp.bool_)
    selected = jnp.zeros((bt, _E), jnp.bool_)
    for _ in range(_KG):
        m = jnp.max(jnp.where(avail, gmb, neg), axis=1, keepdims=True)
        gsel = jnp.min(jnp.where(avail & (gmb == m), gcol, _G), axis=1,
                       keepdims=True)
        hit = gcol == gsel
        selected = selected | hit
        avail = avail & (~hit)

    sm = jnp.where(selected, s, jnp.float32(0.0))
    avail2 = jnp.ones((bt, _E), jnp.bool_)
    wvals = []
    widxs = []
    for _ in range(_K):
        m = jnp.max(jnp.where(avail2, sm, neg), axis=1, keepdims=True)
        idx = jnp.min(jnp.where(avail2 & (sm == m), col, _E), axis=1,
                      keepdims=True)
        hit = col == idx
        wv = jnp.max(jnp.where(hit, s, neg), axis=1, keepdims=True)
        wvals.append(wv)
        widxs.append(idx)
        avail2 = avail2 & (~hit)

    wts = jnp.concatenate(wvals, axis=1)
    idxs = jnp.concatenate(widxs, axis=1)
    wts = wts / jnp.sum(wts, axis=1, keepdims=True)
    w_out_ref[...] = wts
    i_out_ref[...] = idxs


@jax.jit
def kernel(x, weight):
    t = x.shape[0]
    bt = 1024
    grid = (t // bt,)
    w_out, i_out = pl.pallas_call(
        _gate_block,
        grid=grid,
        in_specs=[
            pl.BlockSpec((bt, _DIM), lambda i: (i, 0)),
            pl.BlockSpec((_E, _DIM), lambda i: (0, 0)),
        ],
        out_specs=[
            pl.BlockSpec((bt, _K), lambda i: (i, 0)),
            pl.BlockSpec((bt, _K), lambda i: (i, 0)),
        ],
        out_shape=[
            jax.ShapeDtypeStruct((t, _K), jnp.float32),
            jax.ShapeDtypeStruct((t, _K), jnp.int32),
        ],
        compiler_params=pltpu.CompilerParams(
            dimension_semantics=("parallel",)),
    )(x, weight)
    return w_out, i_out


# speedup vs baseline: 8.4109x; 3.2449x over previous
import jax
import jax.numpy as jnp
from jax.experimental import pallas as pl
from jax.experimental.pallas import tpu as pltpu

_DIM = 2048
_E = 64
_K = 8
_G = 8
_KG = 4
_GS = _E // _G


def _gate_block(x_ref, w_ref, w_out_ref, i_out_ref):
    x = x_ref[...]
    w = w_ref[...]
    logits = jax.lax.dot_general(
        w, x, (((1,), (1,)), ((), ())), preferred_element_type=jnp.float32)
    s = jax.nn.sigmoid(logits)
    bt = s.shape[1]
    neg = jnp.float32(-jnp.inf)

    gms = [jnp.max(s[g * _GS:(g + 1) * _GS], axis=0, keepdims=True)
           for g in range(_G)]

    picked = [jnp.zeros((1, bt), jnp.bool_) for _ in range(_G)]
    for _ in range(_KG):
        mm = jnp.where(picked[0], neg, gms[0])
        for g in range(1, _G):
            mm = jnp.maximum(mm, jnp.where(picked[g], neg, gms[g]))
        gsel = jnp.full((1, bt), jnp.float32(_G))
        for g in reversed(range(_G)):
            gsel = jnp.where((~picked[g]) & (gms[g] == mm),
                             jnp.float32(g), gsel)
        for g in range(_G):
            picked[g] = picked[g] | (gsel == jnp.float32(g))

    sm = jnp.concatenate(
        [jnp.where(picked[g], s[g * _GS:(g + 1) * _GS], jnp.float32(0.0))
         for g in range(_G)], axis=0)

    rowf = jax.lax.broadcasted_iota(jnp.int32, (_E, bt), 0).astype(jnp.float32)
    wvals = []
    widxs = []
    for _ in range(_K):
        m = jnp.max(sm, axis=0, keepdims=True)
        idx = jnp.min(jnp.where(sm == m, rowf, jnp.float32(_E)),
                      axis=0, keepdims=True)
        hit = rowf == idx
        wv = jnp.max(jnp.where(hit, s, neg), axis=0, keepdims=True)
        sm = jnp.where(hit, jnp.float32(-1.0), sm)
        wvals.append(wv)
        widxs.append(idx)

    wts = jnp.concatenate(wvals, axis=0)
    idxs = jnp.concatenate(widxs, axis=0)
    wts = wts / jnp.sum(wts, axis=0, keepdims=True)
    w_out_ref[...] = wts
    i_out_ref[...] = idxs.astype(jnp.int32)


@jax.jit
def kernel(x, weight):
    t = x.shape[0]
    bt = 1024
    grid = (t // bt,)
    w_out, i_out = pl.pallas_call(
        _gate_block,
        grid=grid,
        in_specs=[
            pl.BlockSpec((bt, _DIM), lambda i: (i, 0)),
            pl.BlockSpec((_E, _DIM), lambda i: (0, 0)),
        ],
        out_specs=[
            pl.BlockSpec((_K, bt), lambda i: (0, i)),
            pl.BlockSpec((_K, bt), lambda i: (0, i)),
        ],
        out_shape=[
            jax.ShapeDtypeStruct((_K, t), jnp.float32),
            jax.ShapeDtypeStruct((_K, t), jnp.int32),
        ],
        compiler_params=pltpu.CompilerParams(
            dimension_semantics=("parallel",)),
    )(x, weight)
    return w_out.T, i_out.T


# speedup vs baseline: 9.2935x; 1.1049x over previous
import jax
import jax.numpy as jnp
from jax.experimental import pallas as pl
from jax.experimental.pallas import tpu as pltpu

_DIM = 2048
_E = 64
_K = 8
_G = 8
_KG = 4
_GS = _E // _G


def _route(s, w_out_ref, i_out_ref, lo):
    bt = s.shape[1]
    neg = jnp.float32(-jnp.inf)

    gms = [jnp.max(s[g * _GS:(g + 1) * _GS], axis=0, keepdims=True)
           for g in range(_G)]

    picked = [jnp.zeros((1, bt), jnp.bool_) for _ in range(_G)]
    for _ in range(_KG):
        mm = jnp.where(picked[0], neg, gms[0])
        for g in range(1, _G):
            mm = jnp.maximum(mm, jnp.where(picked[g], neg, gms[g]))
        gsel = jnp.full((1, bt), jnp.float32(_G))
        for g in reversed(range(_G)):
            gsel = jnp.where((~picked[g]) & (gms[g] == mm),
                             jnp.float32(g), gsel)
        for g in range(_G):
            picked[g] = picked[g] | (gsel == jnp.float32(g))

    sm = jnp.concatenate(
        [jnp.where(picked[g], s[g * _GS:(g + 1) * _GS], jnp.float32(0.0))
         for g in range(_G)], axis=0)

    rowf = jax.lax.broadcasted_iota(jnp.int32, (_E, bt), 0).astype(jnp.float32)
    wvals = []
    widxs = []
    for _ in range(_K):
        m = jnp.max(sm, axis=0, keepdims=True)
        idx = jnp.min(jnp.where(sm == m, rowf, jnp.float32(_E)),
                      axis=0, keepdims=True)
        hit = rowf == idx
        wv = jnp.max(jnp.where(hit, s, neg), axis=0, keepdims=True)
        sm = jnp.where(hit, jnp.float32(-1.0), sm)
        wvals.append(wv)
        widxs.append(idx)

    wts = jnp.concatenate(wvals, axis=0)
    idxs = jnp.concatenate(widxs, axis=0)
    wts = wts / jnp.sum(wts, axis=0, keepdims=True)
    w_out_ref[:, lo:lo + bt] = wts
    i_out_ref[:, lo:lo + bt] = idxs.astype(jnp.int32)


def _gate_block(x_ref, w_ref, w_out_ref, i_out_ref):
    w = w_ref[...]
    bt = x_ref.shape[0]
    sub = 1024
    scores = []
    for j in range(bt // sub):
        x = x_ref[j * sub:(j + 1) * sub, :]
        logits = jax.lax.dot_general(
            w, x, (((1,), (1,)), ((), ())),
            preferred_element_type=jnp.float32)
        scores.append(jax.nn.sigmoid(logits))
    for j, s in enumerate(scores):
        _route(s, w_out_ref, i_out_ref, j * sub)


@jax.jit
def kernel(x, weight):
    t = x.shape[0]
    bt = 2048
    grid = (t // bt,)
    w_out, i_out = pl.pallas_call(
        _gate_block,
        grid=grid,
        in_specs=[
            pl.BlockSpec((bt, _DIM), lambda i: (i, 0)),
            pl.BlockSpec((_E, _DIM), lambda i: (0, 0)),
        ],
        out_specs=[
            pl.BlockSpec((_K, bt), lambda i: (0, i)),
            pl.BlockSpec((_K, bt), lambda i: (0, i)),
        ],
        out_shape=[
            jax.ShapeDtypeStruct((_K, t), jnp.float32),
            jax.ShapeDtypeStruct((_K, t), jnp.int32),
        ],
        compiler_params=pltpu.CompilerParams(
            dimension_semantics=("parallel",)),
    )(x, weight)
    return w_out.T, i_out.T


# speedup vs baseline: 9.4754x; 1.0196x over previous
import jax
import jax.numpy as jnp
from jax.experimental import pallas as pl
from jax.experimental.pallas import tpu as pltpu

_DIM = 2048
_E = 64
_K = 8
_G = 8
_KG = 4
_GS = _E // _G


def _route(s, w_out_ref, i_out_ref, lo):
    bt = s.shape[1]
    neg = jnp.float32(-jnp.inf)

    gms = [jnp.max(s[g * _GS:(g + 1) * _GS], axis=0, keepdims=True)
           for g in range(_G)]

    picked = [jnp.zeros((1, bt), jnp.bool_) for _ in range(_G)]
    for _ in range(_KG):
        mm = jnp.where(picked[0], neg, gms[0])
        for g in range(1, _G):
            mm = jnp.maximum(mm, jnp.where(picked[g], neg, gms[g]))
        gsel = jnp.full((1, bt), jnp.float32(_G))
        for g in reversed(range(_G)):
            gsel = jnp.where((~picked[g]) & (gms[g] == mm),
                             jnp.float32(g), gsel)
        for g in range(_G):
            picked[g] = picked[g] | (gsel == jnp.float32(g))

    sm = jnp.concatenate(
        [jnp.where(picked[g], s[g * _GS:(g + 1) * _GS], jnp.float32(0.0))
         for g in range(_G)], axis=0)

    rowf = jax.lax.broadcasted_iota(jnp.int32, (_E, bt), 0).astype(jnp.float32)
    wvals = []
    widxs = []
    for _ in range(_K):
        m = jnp.max(sm, axis=0, keepdims=True)
        idx = jnp.min(jnp.where(sm == m, rowf, jnp.float32(_E)),
                      axis=0, keepdims=True)
        hit = rowf == idx
        sm = jnp.where(hit, jnp.float32(-1.0), sm)
        wvals.append(m)
        widxs.append(idx)

    wts = jnp.concatenate(wvals, axis=0)
    idxs = jnp.concatenate(widxs, axis=0)
    wts = wts / jnp.sum(wts, axis=0, keepdims=True)
    w_out_ref[:, lo:lo + bt] = wts
    i_out_ref[:, lo:lo + bt] = idxs.astype(jnp.int32)


def _gate_block(x_ref, w_ref, w_out_ref, i_out_ref):
    w = w_ref[...]
    bt = x_ref.shape[0]
    sub = 1024
    scores = []
    for j in range(bt // sub):
        x = x_ref[j * sub:(j + 1) * sub, :]
        logits = jax.lax.dot_general(
            w, x, (((1,), (1,)), ((), ())),
            preferred_element_type=jnp.float32)
        scores.append(jax.nn.sigmoid(logits))
    for j, s in enumerate(scores):
        _route(s, w_out_ref, i_out_ref, j * sub)


@jax.jit
def kernel(x, weight):
    t = x.shape[0]
    bt = 2048
    grid = (t // bt,)
    w_out, i_out = pl.pallas_call(
        _gate_block,
        grid=grid,
        in_specs=[
            pl.BlockSpec((bt, _DIM), lambda i: (i, 0)),
            pl.BlockSpec((_E, _DIM), lambda i: (0, 0)),
        ],
        out_specs=[
            pl.BlockSpec((_K, bt), lambda i: (0, i)),
            pl.BlockSpec((_K, bt), lambda i: (0, i)),
        ],
        out_shape=[
            jax.ShapeDtypeStruct((_K, t), jnp.float32),
            jax.ShapeDtypeStruct((_K, t), jnp.int32),
        ],
        compiler_params=pltpu.CompilerParams(
            dimension_semantics=("parallel",)),
    )(x, weight)
    return w_out.T, i_out.T
